# baseline (speedup 1.0000x reference)
import functools

import jax
import jax.numpy as jnp
from jax import lax
from jax.experimental import pallas as pl
from jax.experimental.pallas import tpu as pltpu, tpu_sc as plsc

EMBED = 64
CHUNK = 512
SUB = 128


@functools.cache
def _build(n_total, n_per_w, nc):
    n_chunks = n_per_w // CHUNK
    mesh = plsc.VectorSubcoreMesh(core_axis_name="c", subcore_axis_name="s")

    @functools.partial(
        pl.kernel,
        out_type=jax.ShapeDtypeStruct((n_total, EMBED), jnp.float32),
        mesh=mesh,
        scratch_types=[
            pltpu.VMEM((CHUNK,), jnp.int32),
            pltpu.VMEM((CHUNK, EMBED), jnp.float32),
            pltpu.SemaphoreType.DMA,
        ],
        compiler_params=pltpu.CompilerParams(use_tc_tiling_on_sc=False),
    )
    def emb(x_hbm, table_hbm, out_hbm, idx_v, rows_v, sem):
        wid = lax.axis_index("s") * nc + lax.axis_index("c")
        base = wid * n_per_w

        def body(g, carry):
            off = base + g * CHUNK
            pltpu.sync_copy(x_hbm.at[pl.ds(off, CHUNK)], idx_v)
            copies = [
                pltpu.async_copy(
                    table_hbm.at[idx_v.at[pl.ds(j * SUB, SUB)]],
                    rows_v.at[pl.ds(j * SUB, SUB)],
                    sem,
                )
                for j in range(CHUNK // SUB)
            ]
            for c in copies:
                c.wait()
            pltpu.sync_copy(rows_v, out_hbm.at[pl.ds(off, CHUNK)])
            return carry

        lax.fori_loop(0, n_chunks, body, 0)

    return emb


def kernel(x, table):
    b, l = x.shape
    n_total = b * l
    info = plsc.get_sparse_core_info()
    nw = info.num_cores * info.num_subcores
    n_per_w = n_total // nw
    out = _build(n_total, n_per_w, info.num_cores)(x.reshape(n_total), table)
    return out.reshape(b, l, EMBED)


# speedup vs baseline: 1.0438x; 1.0438x over previous
import functools

import jax
import jax.numpy as jnp
from jax import lax
from jax.experimental import pallas as pl
from jax.experimental.pallas import tpu as pltpu, tpu_sc as plsc

EMBED = 64
CHUNK = 512
SUB = 128
NBUF = 2


@functools.cache
def _build(n_total, n_per_w, nc):
    n_chunks = n_per_w // CHUNK
    assert n_chunks % NBUF == 0
    mesh = plsc.VectorSubcoreMesh(core_axis_name="c", subcore_axis_name="s")

    @functools.partial(
        pl.kernel,
        out_type=jax.ShapeDtypeStruct((n_total, EMBED), jnp.float32),
        mesh=mesh,
        scratch_types=[
            pltpu.VMEM((n_per_w,), jnp.int32),
            pltpu.VMEM((NBUF, CHUNK, EMBED), jnp.float32),
            [pltpu.SemaphoreType.DMA] * NBUF,
            [pltpu.SemaphoreType.DMA] * NBUF,
        ],
        compiler_params=pltpu.CompilerParams(use_tc_tiling_on_sc=False),
    )
    def emb(x_hbm, table_hbm, out_hbm, idx_v, rows_v, gsems, wsems):
        wid = lax.axis_index("s") * nc + lax.axis_index("c")
        base = wid * n_per_w
        pltpu.sync_copy(x_hbm.at[pl.ds(base, n_per_w)], idx_v)

        def fire_gathers(g, b):
            for j in range(CHUNK // SUB):
                pltpu.async_copy(
                    table_hbm.at[idx_v.at[pl.ds(g * CHUNK + j * SUB, SUB)]],
                    rows_v.at[b, pl.ds(j * SUB, SUB)],
                    gsems[b],
                )

        def wait_gathers(b):
            for j in range(CHUNK // SUB):
                pltpu.make_async_copy(
                    table_hbm.at[idx_v.at[pl.ds(j * SUB, SUB)]],
                    rows_v.at[b, pl.ds(j * SUB, SUB)],
                    gsems[b],
                ).wait()

        for b in range(NBUF):
            fire_gathers(b, b)

        def body(g0, carry):
            for b in range(NBUF):
                g = g0 + b
                off = base + g * CHUNK
                wait_gathers(b)
                wb = pltpu.async_copy(
                    rows_v.at[b], out_hbm.at[pl.ds(off, CHUNK)], wsems[b]
                )
                wb.wait()

                @pl.when(g + NBUF < n_chunks)
                def _():
                    fire_gathers(g + NBUF, b)

            return carry

        lax.fori_loop(0, n_chunks // NBUF, lambda i, c: body(i * NBUF, c), 0)

    return emb


def kernel(x, table):
    b, l = x.shape
    n_total = b * l
    info = plsc.get_sparse_core_info()
    nw = info.num_cores * info.num_subcores
    n_per_w = n_total // nw
    out = _build(n_total, n_per_w, info.num_cores)(x.reshape(n_total), table)
    return out.reshape(b, l, EMBED)
